# baseline (speedup 1.0000x reference)
import functools

import jax
import jax.numpy as jnp
from jax.experimental import pallas as pl
from jax.experimental.pallas import tpu as pltpu
from jax.experimental.pallas import tpu_sc as plsc

B, L, H = 4096, 50, 64
BL = B * L
N_DUR = 16
N_WATCH = 32
N_TG = 32
MAX_SEQ_LEN = 50

_OFF_POS = 0
_OFF_DUR = _OFF_POS + MAX_SEQ_LEN
_OFF_WATCH = _OFF_DUR + (N_DUR + 1)
_OFF_TG = _OFF_WATCH + (N_WATCH + 1)
_OFF_WR = _OFF_TG + (N_TG + 1)
_N_ROWS = 136

_BB = 64
_T = _BB * L
_G = B // _BB

_W = 128


_NC, _NS = 2, 16
_NW = _NC * _NS
_BPW = BL // _NW
_CH = 128


_RBLK = 8192


def _widen_rows(item_table):
    rows = item_table.shape[0]
    grid = (rows + _RBLK - 1) // _RBLK

    def body(i_ref, o_ref):
        x = i_ref[...]
        o_ref[...] = jnp.concatenate([x, x], axis=1)

    return pl.pallas_call(
        body,
        grid=(grid,),
        in_specs=[pl.BlockSpec((_RBLK, H), lambda i: (i, 0))],
        out_specs=pl.BlockSpec((_RBLK, 2 * H), lambda i: (i, 0)),
        out_shape=jax.ShapeDtypeStruct((rows, 2 * H), jnp.float32),
    )(item_table)


def _sc_gather_item(table_wide, ids_flat):
    mesh = plsc.VectorSubcoreMesh(core_axis_name="c", subcore_axis_name="s")

    @functools.partial(
        pl.kernel,
        out_type=jax.ShapeDtypeStruct((BL, 2 * H), jnp.float32),
        mesh=mesh,
        scratch_types=[
            pltpu.VMEM((_CH,), jnp.int32),
            pltpu.VMEM((_CH, 2 * H), jnp.float32),
            pltpu.SemaphoreType.DMA,
        ],
    )
    def gather_kernel(tbl_hbm, ids_hbm, out_hbm, idx_v, rows_v, sem):
        wid = jax.lax.axis_index("s") * _NC + jax.lax.axis_index("c")
        base = wid * _BPW

        @pl.loop(0, _BPW, step=_CH)
        def _(off):
            pltpu.sync_copy(ids_hbm.at[pl.ds(base + off, _CH)], idx_v)
            pltpu.async_copy(tbl_hbm.at[idx_v], rows_v, sem).wait()
            pltpu.sync_copy(rows_v, out_hbm.at[pl.ds(base + off, _CH)])

    return gather_kernel(table_wide, ids_flat)


def _tc_body(item_ref, p_ref, d_ref, w_ref, t_ref, wr_ref, tbl_ref,
             bias_ref, g_ref, b_ref, o_ref):
    p = p_ref[0, :, :]
    d = d_ref[0, :, :]
    w = w_ref[0, :, :]
    t = t_ref[0, :, :]
    wr = wr_ref[0, :, :]

    row = jax.lax.broadcasted_iota(jnp.int32, (_N_ROWS, _T), 0)
    cmp = (
        (row == p + _OFF_POS)
        | (row == d + _OFF_DUR)
        | (row == w + _OFF_WATCH)
        | (row == t + _OFF_TG)
    )
    hot = jnp.where(row == _OFF_WR, wr.astype(jnp.bfloat16),
                    cmp.astype(jnp.bfloat16))
    looked = jax.lax.dot_general(
        hot, tbl_ref[...], (((0,), (0,)), ((), ())),
        preferred_element_type=jnp.float32)

    x = item_ref[:, :H] + looked + bias_ref[...]
    avg = jnp.full((H, H), 1.0 / H, jnp.bfloat16)
    mu = jax.lax.dot_general(x.astype(jnp.bfloat16), avg,
                             (((1,), (0,)), ((), ())),
                             preferred_element_type=jnp.float32)
    xc = x - mu
    var = jax.lax.dot_general((xc * xc).astype(jnp.bfloat16), avg,
                              (((1,), (0,)), ((), ())),
                              preferred_element_type=jnp.float32)
    y = xc * jax.lax.rsqrt(var + 1e-5)
    y = y * g_ref[...] + b_ref[...]
    o_ref[...] = y.reshape(_BB, L, H)


def _tc_enrich(item_wide, p3, d3, w3, t3, wr3, tbl, bias, gamma, beta):
    idx_spec = pl.BlockSpec((1, 1, _T), lambda i: (i, 0, 0))
    full = lambda shape: pl.BlockSpec(shape, lambda i: (0,) * len(shape))
    return pl.pallas_call(
        _tc_body,
        grid=(_G,),
        in_specs=[
            pl.BlockSpec((_T, 2 * H), lambda i: (i, 0)),
            idx_spec, idx_spec, idx_spec, idx_spec, idx_spec,
            full((_N_ROWS, H)),
            full((1, H)),
            full((1, H)),
            full((1, H)),
        ],
        out_specs=pl.BlockSpec((_BB, L, H), lambda i: (i, 0, 0)),
        out_shape=jax.ShapeDtypeStruct((B, L, H), jnp.float32),
    )(item_wide, p3, d3, w3, t3, wr3, tbl, bias, gamma, beta)


def kernel(item_ids, positions, watch_ratios, watch_bucket_ids,
           duration_bucket_ids, time_gap_bucket_ids, item_table, pos_table,
           tg_table, dur_table, watch_table, w_dur, b_dur, w_wr, b_wr,
           ln_gamma, ln_beta):
    ids_flat = item_ids.astype(jnp.int32).reshape(BL)
    table_wide = jnp.concatenate([item_table, item_table], axis=1)
    item_wide = _sc_gather_item(table_wide, ids_flat)

    dur_ids = jnp.arange(N_DUR + 1, dtype=jnp.float32)
    dur_tbl2 = dur_table + jnp.log1p(dur_ids)[:, None] * w_dur
    tbl = jnp.zeros((_N_ROWS, H), jnp.float32)
    tbl = tbl.at[_OFF_POS:_OFF_POS + MAX_SEQ_LEN].set(pos_table)
    tbl = tbl.at[_OFF_DUR:_OFF_DUR + N_DUR + 1].set(dur_tbl2)
    tbl = tbl.at[_OFF_WATCH:_OFF_WATCH + N_WATCH + 1].set(watch_table)
    tbl = tbl.at[_OFF_TG:_OFF_TG + N_TG + 1].set(tg_table)
    tbl = tbl.at[_OFF_WR].set(w_wr)
    tbl = tbl.astype(jnp.bfloat16)
    bias = (b_dur + b_wr).reshape(1, H)

    p3 = positions.astype(jnp.int32).reshape(_G, 1, _T)
    d3 = duration_bucket_ids.astype(jnp.int32).reshape(_G, 1, _T)
    w3 = watch_bucket_ids.astype(jnp.int32).reshape(_G, 1, _T)
    t3 = time_gap_bucket_ids.astype(jnp.int32).reshape(_G, 1, _T)
    wr3 = watch_ratios.reshape(_G, 1, _T)

    return _tc_enrich(item_wide, p3, d3, w3, t3, wr3, tbl, bias,
                      ln_gamma.reshape(1, H), ln_beta.reshape(1, H))


# speedup vs baseline: 1.0354x; 1.0354x over previous
import functools

import jax
import jax.numpy as jnp
from jax.experimental import pallas as pl
from jax.experimental.pallas import tpu as pltpu
from jax.experimental.pallas import tpu_sc as plsc

B, L, H = 4096, 50, 64
BL = B * L
N_DUR = 16
N_WATCH = 32
N_TG = 32
MAX_SEQ_LEN = 50

_OFF_POS = 0
_OFF_DUR = _OFF_POS + MAX_SEQ_LEN
_OFF_WATCH = _OFF_DUR + (N_DUR + 1)
_OFF_TG = _OFF_WATCH + (N_WATCH + 1)
_OFF_WR = _OFF_TG + (N_TG + 1)
_N_ROWS = 136

_BB = 64
_T = _BB * L
_G = B // _BB

_W = 128


_NC, _NS = 2, 16
_NW = _NC * _NS
_BPW = BL // _NW
_CH = 128


def _sc_gather_item(table_pairs, phys_ids):
    mesh = plsc.VectorSubcoreMesh(core_axis_name="c", subcore_axis_name="s")

    @functools.partial(
        pl.kernel,
        out_type=jax.ShapeDtypeStruct((BL, 2 * H), jnp.float32),
        mesh=mesh,
        scratch_types=[
            pltpu.VMEM((_CH,), jnp.int32),
            pltpu.VMEM((_CH, 2 * H), jnp.float32),
            pltpu.SemaphoreType.DMA,
        ],
    )
    def gather_kernel(tbl_hbm, ids_hbm, out_hbm, idx_v, rows_v, sem):
        wid = jax.lax.axis_index("s") * _NC + jax.lax.axis_index("c")
        base = wid * _BPW

        @pl.loop(0, _BPW, step=_CH)
        def _(off):
            pltpu.sync_copy(ids_hbm.at[pl.ds(base + off, _CH)], idx_v)
            pltpu.async_copy(tbl_hbm.at[idx_v], rows_v, sem).wait()
            pltpu.sync_copy(rows_v, out_hbm.at[pl.ds(base + off, _CH)])

    return gather_kernel(table_pairs, phys_ids)


def _tc_body(item_ref, id_ref, p_ref, d_ref, w_ref, t_ref, wr_ref, tbl_ref,
             bias_ref, g_ref, b_ref, o_ref):
    p = p_ref[0, :, :]
    d = d_ref[0, :, :]
    w = w_ref[0, :, :]
    t = t_ref[0, :, :]
    wr = wr_ref[0, :, :]
    ids = id_ref[0, :, :]

    row = jax.lax.broadcasted_iota(jnp.int32, (_N_ROWS, _T), 0)
    cmp = (
        (row == p + _OFF_POS)
        | (row == d + _OFF_DUR)
        | (row == w + _OFF_WATCH)
        | (row == t + _OFF_TG)
    )
    hot = jnp.where(row == _OFF_WR, wr.astype(jnp.bfloat16),
                    cmp.astype(jnp.bfloat16))
    looked = jax.lax.dot_general(
        hot, tbl_ref[...], (((0,), (0,)), ((), ())),
        preferred_element_type=jnp.float32)

    ones_row = jnp.full((1, H), 1.0, jnp.bfloat16)
    par = (ids & 1).astype(jnp.bfloat16)
    par64 = jax.lax.dot_general(par, ones_row, (((0,), (0,)), ((), ())),
                                preferred_element_type=jnp.float32)

    pairs = item_ref[...]
    left = pairs[:, :H]
    right = pairs[:, H:]
    item = left + par64 * (right - left)

    x = item + looked + bias_ref[...]
    avg = jnp.full((H, H), 1.0 / H, jnp.bfloat16)
    mu = jax.lax.dot_general(x.astype(jnp.bfloat16), avg,
                             (((1,), (0,)), ((), ())),
                             preferred_element_type=jnp.float32)
    xc = x - mu
    var = jax.lax.dot_general((xc * xc).astype(jnp.bfloat16), avg,
                              (((1,), (0,)), ((), ())),
                              preferred_element_type=jnp.float32)
    y = xc * jax.lax.rsqrt(var + 1e-5)
    y = y * g_ref[...] + b_ref[...]
    o_ref[...] = y.reshape(_BB, L, H)


def _tc_enrich(item_pairs, ids3, p3, d3, w3, t3, wr3, tbl, bias, gamma, beta):
    idx_spec = pl.BlockSpec((1, 1, _T), lambda i: (i, 0, 0))
    full = lambda shape: pl.BlockSpec(shape, lambda i: (0,) * len(shape))
    return pl.pallas_call(
        _tc_body,
        grid=(_G,),
        in_specs=[
            pl.BlockSpec((_T, 2 * H), lambda i: (i, 0)),
            idx_spec, idx_spec, idx_spec, idx_spec, idx_spec, idx_spec,
            full((_N_ROWS, H)),
            full((1, H)),
            full((1, H)),
            full((1, H)),
        ],
        out_specs=pl.BlockSpec((_BB, L, H), lambda i: (i, 0, 0)),
        out_shape=jax.ShapeDtypeStruct((B, L, H), jnp.float32),
    )(item_pairs, ids3, p3, d3, w3, t3, wr3, tbl, bias, gamma, beta)


def kernel(item_ids, positions, watch_ratios, watch_bucket_ids,
           duration_bucket_ids, time_gap_bucket_ids, item_table, pos_table,
           tg_table, dur_table, watch_table, w_dur, b_dur, w_wr, b_wr,
           ln_gamma, ln_beta):
    ids_flat = item_ids.astype(jnp.int32).reshape(BL)
    table_pairs = item_table.reshape(item_table.shape[0] // 2, 2 * H)
    item_pairs = _sc_gather_item(table_pairs, ids_flat >> 1)

    dur_ids = jnp.arange(N_DUR + 1, dtype=jnp.float32)
    dur_tbl2 = dur_table + jnp.log1p(dur_ids)[:, None] * w_dur
    tbl = jnp.zeros((_N_ROWS, H), jnp.float32)
    tbl = tbl.at[_OFF_POS:_OFF_POS + MAX_SEQ_LEN].set(pos_table)
    tbl = tbl.at[_OFF_DUR:_OFF_DUR + N_DUR + 1].set(dur_tbl2)
    tbl = tbl.at[_OFF_WATCH:_OFF_WATCH + N_WATCH + 1].set(watch_table)
    tbl = tbl.at[_OFF_TG:_OFF_TG + N_TG + 1].set(tg_table)
    tbl = tbl.at[_OFF_WR].set(w_wr)
    tbl = tbl.astype(jnp.bfloat16)
    bias = (b_dur + b_wr).reshape(1, H)

    ids3 = ids_flat.reshape(_G, 1, _T)
    p3 = positions.astype(jnp.int32).reshape(_G, 1, _T)
    d3 = duration_bucket_ids.astype(jnp.int32).reshape(_G, 1, _T)
    w3 = watch_bucket_ids.astype(jnp.int32).reshape(_G, 1, _T)
    t3 = time_gap_bucket_ids.astype(jnp.int32).reshape(_G, 1, _T)
    wr3 = watch_ratios.reshape(_G, 1, _T)

    return _tc_enrich(item_pairs, ids3, p3, d3, w3, t3, wr3, tbl, bias,
                      ln_gamma.reshape(1, H), ln_beta.reshape(1, H))


# speedup vs baseline: 1.0661x; 1.0297x over previous
import functools

import jax
import jax.numpy as jnp
from jax.experimental import pallas as pl
from jax.experimental.pallas import tpu as pltpu
from jax.experimental.pallas import tpu_sc as plsc

B, L, H = 4096, 50, 64
BL = B * L
N_DUR = 16
N_WATCH = 32
N_TG = 32
MAX_SEQ_LEN = 50

_OFF_POS = 0
_OFF_DUR = _OFF_POS + MAX_SEQ_LEN
_OFF_WATCH = _OFF_DUR + (N_DUR + 1)
_OFF_TG = _OFF_WATCH + (N_WATCH + 1)
_OFF_WR = _OFF_TG + (N_TG + 1)
_N_ROWS = 136

_BB = 64
_T = _BB * L
_G = B // _BB

_W = 128


_NC, _NS = 2, 16
_NW = _NC * _NS
_BPW = BL // _NW
_CH = 128


def _sc_gather_item(table_pairs, phys_ids):
    mesh = plsc.VectorSubcoreMesh(core_axis_name="c", subcore_axis_name="s")

    @functools.partial(
        pl.kernel,
        out_type=jax.ShapeDtypeStruct((BL, 2 * H), jnp.float32),
        mesh=mesh,
        scratch_types=[
            pltpu.VMEM((_CH,), jnp.int32),
            pltpu.VMEM((_CH,), jnp.int32),
            pltpu.VMEM((_CH, 2 * H), jnp.float32),
            pltpu.VMEM((_CH, 2 * H), jnp.float32),
            pltpu.SemaphoreType.DMA,
            pltpu.SemaphoreType.DMA,
            pltpu.SemaphoreType.DMA,
        ],
    )
    def gather_kernel(tbl_hbm, ids_hbm, out_hbm, idx0, idx1, rows0, rows1,
                      gsem, osem0, osem1):
        wid = jax.lax.axis_index("s") * _NC + jax.lax.axis_index("c")
        base = wid * _BPW

        @pl.loop(0, _BPW, step=2 * _CH)
        def _(off):
            first = off == 0

            pltpu.sync_copy(ids_hbm.at[pl.ds(base + off, _CH)], idx0)

            @pl.when(jnp.logical_not(first))
            def _():
                pltpu.make_async_copy(
                    rows0, out_hbm.at[pl.ds(base + off - _CH, _CH)],
                    osem0).wait()

            pltpu.async_copy(tbl_hbm.at[idx0], rows0, gsem).wait()
            pltpu.make_async_copy(
                rows0, out_hbm.at[pl.ds(base + off, _CH)], osem0).start()

            pltpu.sync_copy(ids_hbm.at[pl.ds(base + off + _CH, _CH)], idx1)

            @pl.when(jnp.logical_not(first))
            def _():
                pltpu.make_async_copy(
                    rows1, out_hbm.at[pl.ds(base + off, _CH)], osem1).wait()

            pltpu.async_copy(tbl_hbm.at[idx1], rows1, gsem).wait()
            pltpu.make_async_copy(
                rows1, out_hbm.at[pl.ds(base + off + _CH, _CH)], osem1).start()

        pltpu.make_async_copy(
            rows0, out_hbm.at[pl.ds(base, _CH)], osem0).wait()
        pltpu.make_async_copy(
            rows1, out_hbm.at[pl.ds(base, _CH)], osem1).wait()

    return gather_kernel(table_pairs, phys_ids)


def _tc_body(item_ref, id_ref, p_ref, d_ref, w_ref, t_ref, wr_ref, tbl_ref,
             bias_ref, g_ref, b_ref, o_ref):
    p = p_ref[0, :, :]
    d = d_ref[0, :, :]
    w = w_ref[0, :, :]
    t = t_ref[0, :, :]
    wr = wr_ref[0, :, :]
    ids = id_ref[0, :, :]

    row = jax.lax.broadcasted_iota(jnp.int32, (_N_ROWS, _T), 0)
    cmp = (
        (row == p + _OFF_POS)
        | (row == d + _OFF_DUR)
        | (row == w + _OFF_WATCH)
        | (row == t + _OFF_TG)
    )
    hot = jnp.where(row == _OFF_WR, wr.astype(jnp.bfloat16),
                    cmp.astype(jnp.bfloat16))
    looked = jax.lax.dot_general(
        hot, tbl_ref[...], (((0,), (0,)), ((), ())),
        preferred_element_type=jnp.float32)

    ones_row = jnp.full((1, H), 1.0, jnp.bfloat16)
    par = (ids & 1).astype(jnp.bfloat16)
    par64 = jax.lax.dot_general(par, ones_row, (((0,), (0,)), ((), ())),
                                preferred_element_type=jnp.float32)

    pairs = item_ref[...]
    left = pairs[:, :H]
    right = pairs[:, H:]
    item = left + par64 * (right - left)

    x = item + looked + bias_ref[...]
    avg = jnp.full((H, H), 1.0 / H, jnp.bfloat16)
    mu = jax.lax.dot_general(x.astype(jnp.bfloat16), avg,
                             (((1,), (0,)), ((), ())),
                             preferred_element_type=jnp.float32)
    xc = x - mu
    var = jax.lax.dot_general((xc * xc).astype(jnp.bfloat16), avg,
                              (((1,), (0,)), ((), ())),
                              preferred_element_type=jnp.float32)
    y = xc * jax.lax.rsqrt(var + 1e-5)
    y = y * g_ref[...] + b_ref[...]
    o_ref[...] = y.reshape(_BB, L, H)


def _tc_enrich(item_pairs, ids3, p3, d3, w3, t3, wr3, tbl, bias, gamma, beta):
    idx_spec = pl.BlockSpec((1, 1, _T), lambda i: (i, 0, 0))
    full = lambda shape: pl.BlockSpec(shape, lambda i: (0,) * len(shape))
    return pl.pallas_call(
        _tc_body,
        grid=(_G,),
        in_specs=[
            pl.BlockSpec((_T, 2 * H), lambda i: (i, 0)),
            idx_spec, idx_spec, idx_spec, idx_spec, idx_spec, idx_spec,
            full((_N_ROWS, H)),
            full((1, H)),
            full((1, H)),
            full((1, H)),
        ],
        out_specs=pl.BlockSpec((_BB, L, H), lambda i: (i, 0, 0)),
        out_shape=jax.ShapeDtypeStruct((B, L, H), jnp.float32),
    )(item_pairs, ids3, p3, d3, w3, t3, wr3, tbl, bias, gamma, beta)


def kernel(item_ids, positions, watch_ratios, watch_bucket_ids,
           duration_bucket_ids, time_gap_bucket_ids, item_table, pos_table,
           tg_table, dur_table, watch_table, w_dur, b_dur, w_wr, b_wr,
           ln_gamma, ln_beta):
    ids_flat = item_ids.astype(jnp.int32).reshape(BL)
    table_pairs = item_table.reshape(item_table.shape[0] // 2, 2 * H)
    item_pairs = _sc_gather_item(table_pairs, ids_flat >> 1)

    dur_ids = jnp.arange(N_DUR + 1, dtype=jnp.float32)
    dur_tbl2 = dur_table + jnp.log1p(dur_ids)[:, None] * w_dur
    tbl = jnp.zeros((_N_ROWS, H), jnp.float32)
    tbl = tbl.at[_OFF_POS:_OFF_POS + MAX_SEQ_LEN].set(pos_table)
    tbl = tbl.at[_OFF_DUR:_OFF_DUR + N_DUR + 1].set(dur_tbl2)
    tbl = tbl.at[_OFF_WATCH:_OFF_WATCH + N_WATCH + 1].set(watch_table)
    tbl = tbl.at[_OFF_TG:_OFF_TG + N_TG + 1].set(tg_table)
    tbl = tbl.at[_OFF_WR].set(w_wr)
    tbl = tbl.astype(jnp.bfloat16)
    bias = (b_dur + b_wr).reshape(1, H)

    ids3 = ids_flat.reshape(_G, 1, _T)
    p3 = positions.astype(jnp.int32).reshape(_G, 1, _T)
    d3 = duration_bucket_ids.astype(jnp.int32).reshape(_G, 1, _T)
    w3 = watch_bucket_ids.astype(jnp.int32).reshape(_G, 1, _T)
    t3 = time_gap_bucket_ids.astype(jnp.int32).reshape(_G, 1, _T)
    wr3 = watch_ratios.reshape(_G, 1, _T)

    return _tc_enrich(item_pairs, ids3, p3, d3, w3, t3, wr3, tbl, bias,
                      ln_gamma.reshape(1, H), ln_beta.reshape(1, H))
